# baseline (speedup 1.0000x reference)
import functools

import jax
import jax.numpy as jnp
from jax import lax
from jax.experimental import pallas as pl
from jax.experimental.pallas import tpu as pltpu
from jax.experimental.pallas import tpu_sc as plsc

B = 8
K = 4096
C = 256
N = 512

_NC, _NS = 2, 16
_NW = _NC * _NS
_ROWS = B * C
_RPW = _ROWS // _NW
_TPB = _NW // B

_GC = 4
_NG = _RPW // _GC
_CHUNK = 128
_CPG = _GC * N // _CHUNK


_NG_FPS = 2
_BG = B // _NG_FPS
_SPB = 8 // _BG
_KG = K // _SPB


def _fps_body(x_ref, y_ref, z_ref, inds_ref, nxyz_ref):
    rows = lax.broadcasted_iota(jnp.int32, (8, 1), 0)
    low = rows < _BG
    rp = lax.broadcasted_iota(jnp.int32, (8, _KG), 0) // _BG * _KG
    iota_g = lax.broadcasted_iota(jnp.int32, (8, _KG), 1) + rp
    lane_n = lax.broadcasted_iota(jnp.int32, (8, N), 1)
    first = lane_n == 0

    def paircomb(v, op):
        if _SPB == 1:
            return v
        return op(v, jnp.roll(v, _BG, axis=0))

    def dup_even(v):
        if _SPB == 1:
            return v
        return jnp.where(low, v, jnp.roll(v, _BG, axis=0))

    xs, ys, zs, inits = [], [], [], []
    for g in range(_NG_FPS):
        xg = x_ref[g]
        yg = y_ref[g]
        zg = z_ref[g]
        xs.append(xg)
        ys.append(yg)
        zs.append(zg)
        lx = dup_even(xg[:, 0:1])
        ly = dup_even(yg[:, 0:1])
        lz = dup_even(zg[:, 0:1])
        dists0 = jnp.full((8, _KG), 1e10, dtype=jnp.float32)
        acc0 = jnp.zeros((8, N), jnp.float32)
        ax0 = jnp.where(first, lx, 0.0)
        ay0 = jnp.where(first, ly, 0.0)
        az0 = jnp.where(first, lz, 0.0)
        inits.append((dists0, lx, ly, lz, acc0, ax0, ay0, az0))

    def body(i, carry):
        out = []
        for g in range(_NG_FPS):
            dists, lx, ly, lz, acc, ax, ay, az = carry[g]
            x, y, z = xs[g], ys[g], zs[g]
            dx = x - lx
            dy = y - ly
            dz = z - lz
            d = dx * dx + dy * dy + dz * dz
            dists = jnp.minimum(dists, d)
            m = jnp.max(dists, axis=1, keepdims=True)
            mm = paircomb(m, jnp.maximum)
            iv = jnp.min(jnp.where(dists == mm, iota_g, K), axis=1,
                         keepdims=True)
            idx = paircomb(iv, jnp.minimum)
            sel = iota_g == idx
            lx = paircomb(jnp.sum(jnp.where(sel, x, 0.0), axis=1,
                                  keepdims=True), jnp.add)
            ly = paircomb(jnp.sum(jnp.where(sel, y, 0.0), axis=1,
                                  keepdims=True), jnp.add)
            lz = paircomb(jnp.sum(jnp.where(sel, z, 0.0), axis=1,
                                  keepdims=True), jnp.add)
            hit = lane_n == i
            acc = jnp.where(hit, idx.astype(jnp.float32), acc)
            ax = jnp.where(hit, lx, ax)
            ay = jnp.where(hit, ly, ay)
            az = jnp.where(hit, lz, az)
            out.append((dists, lx, ly, lz, acc, ax, ay, az))
        return tuple(out)

    carry = lax.fori_loop(1, N, body, tuple(inits))

    def compact(a):
        return a[:_BG]

    accs = [jnp.concatenate([compact(carry[g][4 + k]) for g in range(_NG_FPS)],
                            axis=0) for k in range(4)]
    inds_ref[...] = accs[0].astype(jnp.int32)
    nxyz_ref[...] = jnp.stack([accs[1], accs[2], accs[3]],
                              axis=-1).reshape(B, 3 * N)


def _regroup(a):
    return (a.reshape(_NG_FPS, _BG, _SPB, _KG)
            .transpose(0, 2, 1, 3).reshape(_NG_FPS, 8, _KG))


@jax.jit
def _fps(x, y, z):
    xg = _regroup(x)
    yg = _regroup(y)
    zg = _regroup(z)
    return pl.pallas_call(
        _fps_body,
        out_shape=(
            jax.ShapeDtypeStruct((B, N), jnp.int32),
            jax.ShapeDtypeStruct((B, 3 * N), jnp.float32),
        ),
    )(xg, yg, zg)


def _gather_body(feat_hbm, inds_hbm, out_hbm,
                 inds_v, idxA, idxB, rowA, rowB, semA, semB):
    wid = lax.axis_index("s") * _NC + lax.axis_index("c")
    b = wid // _TPB
    q = wid % _TPB
    c0 = q * _RPW

    pltpu.sync_copy(inds_hbm.at[pl.ds(b * N, N)], inds_v)

    base0 = b * (C * K) + c0 * K

    def build(idxbuf, g):
        for j in range(_CPG):
            base = base0 + (g * _GC + j // 4) * K
            for s in range(8):
                idxbuf[j, pl.ds(s * 16, 16)] = (
                    inds_v[pl.ds((j % 4) * _CHUNK + s * 16, 16)] + base)

    def fire(idxbuf, rowbuf, sem):
        for j in range(_CPG):
            pltpu.async_copy(feat_hbm.at[idxbuf.at[j]],
                             rowbuf.at[pl.ds(j * _CHUNK, _CHUNK)], sem)

    def drain(rowbuf, sem):
        pltpu.make_async_copy(feat_hbm.at[pl.ds(0, _GC * N)], rowbuf, sem).wait()

    def out(rowbuf, g):
        pltpu.sync_copy(rowbuf,
                        out_hbm.at[pl.ds((b * C + c0 + g * _GC) * N, _GC * N)])

    build(idxA, 0)
    fire(idxA, rowA, semA)

    def body(it, carry):
        gA = 2 * it
        build(idxB, gA + 1)
        fire(idxB, rowB, semB)
        drain(rowA, semA)
        out(rowA, gA)

        @pl.when(it < _NG // 2 - 1)
        def _():
            build(idxA, gA + 2)
            fire(idxA, rowA, semA)

        drain(rowB, semB)
        out(rowB, gA + 1)
        return carry

    lax.fori_loop(0, _NG // 2, body, 0)


@functools.cache
def _sc_gather_fn():
    return pl.kernel(
        _gather_body,
        out_type=jax.ShapeDtypeStruct((B * C * N,), jnp.float32),
        mesh=plsc.VectorSubcoreMesh(core_axis_name="c", subcore_axis_name="s"),
        scratch_types=[
            pltpu.VMEM((N,), jnp.int32),
            pltpu.VMEM((_CPG, _CHUNK), jnp.int32),
            pltpu.VMEM((_CPG, _CHUNK), jnp.int32),
            pltpu.VMEM((_GC * N,), jnp.float32),
            pltpu.VMEM((_GC * N,), jnp.float32),
            pltpu.SemaphoreType.DMA,
            pltpu.SemaphoreType.DMA,
        ],
    )


@jax.jit
def kernel(xyz, features):
    x = xyz[:, :, 0]
    y = xyz[:, :, 1]
    z = xyz[:, :, 2]
    sample_inds, nxyz = _fps(x, y, z)
    out_flat = _sc_gather_fn()(features.reshape(-1), sample_inds.reshape(-1))
    new_features = out_flat.reshape(B, C, N)
    new_xyz = nxyz.reshape(B, N, 3)
    return new_xyz, new_features, sample_inds


# speedup vs baseline: 1.2829x; 1.2829x over previous
import functools

import jax
import jax.numpy as jnp
from jax import lax
from jax.experimental import pallas as pl
from jax.experimental.pallas import tpu as pltpu
from jax.experimental.pallas import tpu_sc as plsc

B = 8
K = 4096
C = 256
N = 512

_NC, _NS = 2, 16
_NW = _NC * _NS
_ROWS = B * C
_RPW = _ROWS // _NW
_TPB = _NW // B

_GC = 4
_NG = _RPW // _GC
_CHUNK = 128
_CPG = _GC * N // _CHUNK


_NG_FPS = 1
_BG = B // _NG_FPS
_SPB = 8 // _BG
_KG = K // _SPB


def _fps_body(x_ref, y_ref, z_ref, inds_ref, nxyz_ref):
    rows = lax.broadcasted_iota(jnp.int32, (8, 1), 0)
    low = rows < _BG
    rp = lax.broadcasted_iota(jnp.int32, (8, _KG), 0) // _BG * _KG
    iota_g = lax.broadcasted_iota(jnp.int32, (8, _KG), 1) + rp
    lane_n = lax.broadcasted_iota(jnp.int32, (8, N), 1)
    first = lane_n == 0

    def paircomb(v, op):
        if _SPB == 1:
            return v
        return op(v, jnp.roll(v, _BG, axis=0))

    def dup_even(v):
        if _SPB == 1:
            return v
        return jnp.where(low, v, jnp.roll(v, _BG, axis=0))

    xs, ys, zs, inits = [], [], [], []
    for g in range(_NG_FPS):
        xg = x_ref[g]
        yg = y_ref[g]
        zg = z_ref[g]
        xs.append(xg)
        ys.append(yg)
        zs.append(zg)
        lx = dup_even(xg[:, 0:1])
        ly = dup_even(yg[:, 0:1])
        lz = dup_even(zg[:, 0:1])
        dists0 = jnp.full((8, _KG), 1e10, dtype=jnp.float32)
        acc0 = jnp.zeros((8, N), jnp.float32)
        ax0 = jnp.where(first, lx, 0.0)
        ay0 = jnp.where(first, ly, 0.0)
        az0 = jnp.where(first, lz, 0.0)
        inits.append((dists0, lx, ly, lz, acc0, ax0, ay0, az0))

    def body(i, carry):
        out = []
        for g in range(_NG_FPS):
            dists, lx, ly, lz, acc, ax, ay, az = carry[g]
            x, y, z = xs[g], ys[g], zs[g]
            dx = x - lx
            dy = y - ly
            dz = z - lz
            d = dx * dx + dy * dy + dz * dz
            dists = jnp.minimum(dists, d)
            m = jnp.max(dists, axis=1, keepdims=True)
            mm = paircomb(m, jnp.maximum)
            iv = jnp.min(jnp.where(dists == mm, iota_g, K), axis=1,
                         keepdims=True)
            idx = paircomb(iv, jnp.minimum)
            sel = iota_g == idx
            lx = paircomb(jnp.sum(jnp.where(sel, x, 0.0), axis=1,
                                  keepdims=True), jnp.add)
            ly = paircomb(jnp.sum(jnp.where(sel, y, 0.0), axis=1,
                                  keepdims=True), jnp.add)
            lz = paircomb(jnp.sum(jnp.where(sel, z, 0.0), axis=1,
                                  keepdims=True), jnp.add)
            hit = lane_n == i
            acc = jnp.where(hit, idx.astype(jnp.float32), acc)
            ax = jnp.where(hit, lx, ax)
            ay = jnp.where(hit, ly, ay)
            az = jnp.where(hit, lz, az)
            out.append((dists, lx, ly, lz, acc, ax, ay, az))
        return tuple(out)

    carry = lax.fori_loop(1, N, body, tuple(inits))

    def compact(a):
        return a[:_BG]

    accs = [jnp.concatenate([compact(carry[g][4 + k]) for g in range(_NG_FPS)],
                            axis=0) for k in range(4)]
    inds_ref[...] = accs[0].astype(jnp.int32)
    nxyz_ref[...] = jnp.stack([accs[1], accs[2], accs[3]],
                              axis=-1).reshape(B, 3 * N)


def _regroup(a):
    return (a.reshape(_NG_FPS, _BG, _SPB, _KG)
            .transpose(0, 2, 1, 3).reshape(_NG_FPS, 8, _KG))


@jax.jit
def _fps(x, y, z):
    xg = _regroup(x)
    yg = _regroup(y)
    zg = _regroup(z)
    return pl.pallas_call(
        _fps_body,
        out_shape=(
            jax.ShapeDtypeStruct((B, N), jnp.int32),
            jax.ShapeDtypeStruct((B, 3 * N), jnp.float32),
        ),
    )(xg, yg, zg)


def _gather_body(feat_hbm, inds_hbm, out_hbm,
                 inds_v, idxA, idxB, rowA, rowB, semA, semB):
    wid = lax.axis_index("s") * _NC + lax.axis_index("c")
    b = wid // _TPB
    q = wid % _TPB
    c0 = q * _RPW

    pltpu.sync_copy(inds_hbm.at[pl.ds(b * N, N)], inds_v)

    base0 = b * (C * K) + c0 * K

    def build(idxbuf, g):
        for j in range(_CPG):
            base = base0 + (g * _GC + j // 4) * K
            for s in range(8):
                idxbuf[j, pl.ds(s * 16, 16)] = (
                    inds_v[pl.ds((j % 4) * _CHUNK + s * 16, 16)] + base)

    def fire(idxbuf, rowbuf, sem):
        for j in range(_CPG):
            pltpu.async_copy(feat_hbm.at[idxbuf.at[j]],
                             rowbuf.at[pl.ds(j * _CHUNK, _CHUNK)], sem)

    def drain(rowbuf, sem):
        pltpu.make_async_copy(feat_hbm.at[pl.ds(0, _GC * N)], rowbuf, sem).wait()

    def out(rowbuf, g):
        pltpu.sync_copy(rowbuf,
                        out_hbm.at[pl.ds((b * C + c0 + g * _GC) * N, _GC * N)])

    build(idxA, 0)
    fire(idxA, rowA, semA)

    def body(it, carry):
        gA = 2 * it
        build(idxB, gA + 1)
        fire(idxB, rowB, semB)
        drain(rowA, semA)
        out(rowA, gA)

        @pl.when(it < _NG // 2 - 1)
        def _():
            build(idxA, gA + 2)
            fire(idxA, rowA, semA)

        drain(rowB, semB)
        out(rowB, gA + 1)
        return carry

    lax.fori_loop(0, _NG // 2, body, 0)


@functools.cache
def _sc_gather_fn():
    return pl.kernel(
        _gather_body,
        out_type=jax.ShapeDtypeStruct((B * C * N,), jnp.float32),
        mesh=plsc.VectorSubcoreMesh(core_axis_name="c", subcore_axis_name="s"),
        scratch_types=[
            pltpu.VMEM((N,), jnp.int32),
            pltpu.VMEM((_CPG, _CHUNK), jnp.int32),
            pltpu.VMEM((_CPG, _CHUNK), jnp.int32),
            pltpu.VMEM((_GC * N,), jnp.float32),
            pltpu.VMEM((_GC * N,), jnp.float32),
            pltpu.SemaphoreType.DMA,
            pltpu.SemaphoreType.DMA,
        ],
    )


@jax.jit
def kernel(xyz, features):
    x = xyz[:, :, 0]
    y = xyz[:, :, 1]
    z = xyz[:, :, 2]
    sample_inds, nxyz = _fps(x, y, z)
    out_flat = _sc_gather_fn()(features.reshape(-1), sample_inds.reshape(-1))
    new_features = out_flat.reshape(B, C, N)
    new_xyz = nxyz.reshape(B, N, 3)
    return new_xyz, new_features, sample_inds
